# baseline (speedup 1.0000x reference)
import functools

import jax
import jax.numpy as jnp
from jax.experimental import pallas as pl
from jax.experimental.pallas import tpu as pltpu

_BT = 512
_NBUF = 12


def _router_body(x_hbm, wt_ref, b_ref, o_ref, bufs, sems, n_chunks):
    def start_copy(c):
        pltpu.make_async_copy(
            x_hbm.at[pl.ds(c * _BT, _BT), :],
            bufs.at[c % _NBUF],
            sems.at[c % _NBUF],
        ).start()

    for c in range(min(_NBUF - 1, n_chunks)):
        start_copy(c)

    for c in range(n_chunks):
        slot = c % _NBUF
        pltpu.make_async_copy(
            x_hbm.at[pl.ds(c * _BT, _BT), :],
            bufs.at[slot],
            sems.at[slot],
        ).wait()
        if c + _NBUF - 1 < n_chunks:
            start_copy(c + _NBUF - 1)
        logits = jnp.dot(bufs[slot], wt_ref[...],
                         preferred_element_type=jnp.float32) + b_ref[...]
        m = jnp.max(logits, axis=-1, keepdims=True)
        e = jnp.exp(logits - m)
        o_ref[pl.ds(c * _BT, _BT), :] = e / jnp.sum(e, axis=-1, keepdims=True)


@jax.jit
def kernel(x, W, b):
    n_tokens, embed_dim = x.shape
    n_experts = W.shape[0]
    wt = W.T
    b2 = b.reshape(1, n_experts)
    n_chunks = n_tokens // _BT
    return pl.pallas_call(
        functools.partial(_router_body, n_chunks=n_chunks),
        in_specs=[
            pl.BlockSpec(memory_space=pltpu.MemorySpace.HBM),
            pl.BlockSpec(memory_space=pltpu.MemorySpace.VMEM),
            pl.BlockSpec(memory_space=pltpu.MemorySpace.VMEM),
        ],
        out_specs=pl.BlockSpec(memory_space=pltpu.MemorySpace.VMEM),
        out_shape=jax.ShapeDtypeStruct((n_tokens, n_experts), jnp.float32),
        scratch_shapes=[
            pltpu.VMEM((_NBUF, _BT, embed_dim), jnp.float32),
            pltpu.SemaphoreType.DMA((_NBUF,)),
        ],
    )(x, wt, b2)
